# baseline (device time: 21956 ns/iter reference)
import jax
import jax.numpy as jnp
from jax import lax
from jax.experimental import pallas as pl
from jax.experimental.pallas import tpu as pltpu

B, S, N = 4, 512, 1024
H, D = 8, 64
K = H * D
S_HALF = S // 2
QR = 64


def kernel(O, Wo):
    O_t = jnp.transpose(O, (0, 2, 3, 1)).reshape(B, K, S)
    O_t = pltpu.with_memory_space_constraint(O_t, pltpu.MemorySpace.HBM)
    Wo = pltpu.with_memory_space_constraint(Wo, pltpu.MemorySpace.HBM)

    def body(o_hbm, wo_hbm, out_hbm, wo_ref, o1, o2, acc,
             ysend, yrecv, xrecv, zrecv, drecv,
             wo_sem, o1_sems, o2_sems, out_sems,
             ys_sems, yr_sems, xs_sems, xr_sems, zs_sems, zr_sems,
             ds_sems, dr_sems):
        my_x = lax.axis_index("x")
        my_y = lax.axis_index("y")
        my_z = lax.axis_index("z")
        ypartner = (my_x, 1 - my_y, my_z)
        xneighbor = (1 - my_x, my_y, my_z)
        zneighbor = (my_x, my_y, 1 - my_z)

        my_start0 = my_y * S_HALF
        other_start0 = (1 - my_y) * S_HALF

        wo_copy = pltpu.make_async_copy(wo_hbm, wo_ref, wo_sem)
        wo_copy.start()
        o1_copies, o2_copies = [], []
        for b in range(B):
            c = pltpu.make_async_copy(
                o_hbm.at[b, :, pl.ds(other_start0 + my_x * 2 * QR, 2 * QR)],
                o1.at[b], o1_sems.at[b],
            )
            c.start()
            o1_copies.append(c)
        for b in range(B):
            c = pltpu.make_async_copy(
                o_hbm.at[b, :, pl.ds(my_start0, S_HALF)],
                o2.at[b], o2_sems.at[b],
            )
            c.start()
            o2_copies.append(c)

        barrier = pltpu.get_barrier_semaphore()
        for nbr in (ypartner, xneighbor, zneighbor):
            pl.semaphore_signal(
                barrier, inc=1, device_id=nbr,
                device_id_type=pl.DeviceIdType.MESH,
            )
        pl.semaphore_wait(barrier, 3)

        wo_copy.wait()

        q_mine = (2 * my_x + my_z) * QR
        q_x = (2 * (1 - my_x) + my_z) * QR
        q_z = (2 * my_x + (1 - my_z)) * QR
        q_d = (2 * (1 - my_x) + (1 - my_z)) * QR

        def _dot(src_ref, b):
            return lax.dot_general(
                src_ref[b], wo_ref[:, :],
                (((0,), (0,)), ((), ())),
                preferred_element_type=jnp.float32,
            )

        y_rdmas = []
        for b in range(B):
            o1_copies[b].wait()
            part = _dot(o1, b)
            ysend[b] = part.astype(jnp.bfloat16)
            r = pltpu.make_async_remote_copy(
                src_ref=ysend.at[b, pl.ds(my_z * QR, QR), :],
                dst_ref=yrecv.at[b],
                send_sem=ys_sems.at[b],
                recv_sem=yr_sems.at[b],
                device_id=ypartner,
                device_id_type=pl.DeviceIdType.MESH,
            )
            r.start()
            y_rdmas.append(r)

        x_rdmas, z_rdmas = [], []
        for b in range(B):
            o2_copies[b].wait()
            acc[b] = _dot(o2, b)
            y_rdmas[b].wait_recv()
            xr = pltpu.make_async_remote_copy(
                src_ref=yrecv.at[b], dst_ref=xrecv.at[b],
                send_sem=xs_sems.at[b], recv_sem=xr_sems.at[b],
                device_id=xneighbor, device_id_type=pl.DeviceIdType.MESH,
            )
            xr.start()
            x_rdmas.append(xr)
            zr = pltpu.make_async_remote_copy(
                src_ref=yrecv.at[b], dst_ref=zrecv.at[b],
                send_sem=zs_sems.at[b], recv_sem=zr_sems.at[b],
                device_id=zneighbor, device_id_type=pl.DeviceIdType.MESH,
            )
            zr.start()
            z_rdmas.append(zr)
            acc[b, pl.ds(q_mine, QR), :] = (
                acc[b, pl.ds(q_mine, QR), :] + yrecv[b].astype(jnp.float32)
            )

        d_rdmas = []
        for b in range(B):
            x_rdmas[b].wait_recv()
            z_rdmas[b].wait_recv()
            if b % 2 == 0:
                dr = pltpu.make_async_remote_copy(
                    src_ref=zrecv.at[b], dst_ref=drecv.at[b],
                    send_sem=ds_sems.at[b], recv_sem=dr_sems.at[b],
                    device_id=xneighbor, device_id_type=pl.DeviceIdType.MESH,
                )
            else:
                dr = pltpu.make_async_remote_copy(
                    src_ref=xrecv.at[b], dst_ref=drecv.at[b],
                    send_sem=ds_sems.at[b], recv_sem=dr_sems.at[b],
                    device_id=zneighbor, device_id_type=pl.DeviceIdType.MESH,
                )
            dr.start()
            d_rdmas.append(dr)
            acc[b, pl.ds(q_x, QR), :] = (
                acc[b, pl.ds(q_x, QR), :] + xrecv[b].astype(jnp.float32)
            )
            acc[b, pl.ds(q_z, QR), :] = (
                acc[b, pl.ds(q_z, QR), :] + zrecv[b].astype(jnp.float32)
            )

        out_copies = []
        for b in range(B):
            d_rdmas[b].wait_recv()
            acc[b, pl.ds(q_d, QR), :] = (
                acc[b, pl.ds(q_d, QR), :] + drecv[b].astype(jnp.float32)
            )
            oc = pltpu.make_async_copy(acc.at[b], out_hbm.at[b], out_sems.at[b])
            oc.start()
            out_copies.append(oc)

        for c in out_copies:
            c.wait()
        for b in range(B):
            y_rdmas[b].wait_send()
            x_rdmas[b].wait_send()
            z_rdmas[b].wait_send()
            d_rdmas[b].wait_send()

    return pl.pallas_call(
        body,
        out_shape=jax.ShapeDtypeStruct((B, S_HALF, N), jnp.float32),
        in_specs=[
            pl.BlockSpec(memory_space=pltpu.MemorySpace.HBM),
            pl.BlockSpec(memory_space=pltpu.MemorySpace.HBM),
        ],
        out_specs=pl.BlockSpec(memory_space=pltpu.MemorySpace.HBM),
        scratch_shapes=[
            pltpu.VMEM((K, N), jnp.float32),
            pltpu.VMEM((B, K, 2 * QR), jnp.float32),
            pltpu.VMEM((B, K, S_HALF), jnp.float32),
            pltpu.VMEM((B, S_HALF, N), jnp.float32),
            pltpu.VMEM((B, 2 * QR, N), jnp.bfloat16),
            pltpu.VMEM((B, QR, N), jnp.bfloat16),
            pltpu.VMEM((B, QR, N), jnp.bfloat16),
            pltpu.VMEM((B, QR, N), jnp.bfloat16),
            pltpu.VMEM((B, QR, N), jnp.bfloat16),
            pltpu.SemaphoreType.DMA,
            pltpu.SemaphoreType.DMA((B,)),
            pltpu.SemaphoreType.DMA((B,)),
            pltpu.SemaphoreType.DMA((B,)),
            pltpu.SemaphoreType.DMA((B,)),
            pltpu.SemaphoreType.DMA((B,)),
            pltpu.SemaphoreType.DMA((B,)),
            pltpu.SemaphoreType.DMA((B,)),
            pltpu.SemaphoreType.DMA((B,)),
            pltpu.SemaphoreType.DMA((B,)),
            pltpu.SemaphoreType.DMA((B,)),
            pltpu.SemaphoreType.DMA((B,)),
        ],
        compiler_params=pltpu.CompilerParams(collective_id=0),
    )(O_t, Wo)


# device time: 21924 ns/iter; 1.0015x vs baseline; 1.0015x over previous
import jax
import jax.numpy as jnp
from jax import lax
from jax.experimental import pallas as pl
from jax.experimental.pallas import tpu as pltpu

B, S, N = 4, 512, 1024
H, D = 8, 64
K = H * D
S_HALF = S // 2
QR = 64


def kernel(O, Wo):
    O_t = jnp.transpose(O, (0, 2, 3, 1)).reshape(B, K, S)
    O_t = pltpu.with_memory_space_constraint(O_t, pltpu.MemorySpace.HBM)
    Wo = pltpu.with_memory_space_constraint(Wo, pltpu.MemorySpace.HBM)

    def body(o_hbm, wo_hbm, out_hbm, wo_ref, o1, o2, acc,
             ysend, yrecv, xrecv, zrecv, drecv,
             wo_sem, o1_sems, o2_sems, out_sems,
             ys_sems, yr_sems, xs_sems, xr_sems, zs_sems, zr_sems,
             ds_sems, dr_sems):
        my_x = lax.axis_index("x")
        my_y = lax.axis_index("y")
        my_z = lax.axis_index("z")
        ypartner = (my_x, 1 - my_y, my_z)
        xneighbor = (1 - my_x, my_y, my_z)
        zneighbor = (my_x, my_y, 1 - my_z)

        my_start0 = my_y * S_HALF
        other_start0 = (1 - my_y) * S_HALF

        wo_copy = pltpu.make_async_copy(wo_hbm, wo_ref, wo_sem)
        wo_copy.start()
        o1_copies, o2_copies = [], []
        for b in range(B):
            c = pltpu.make_async_copy(
                o_hbm.at[b, :, pl.ds(other_start0 + my_x * 2 * QR, 2 * QR)],
                o1.at[b], o1_sems.at[b],
            )
            c.start()
            o1_copies.append(c)
        for b in range(B):
            c = pltpu.make_async_copy(
                o_hbm.at[b, :, pl.ds(my_start0, S_HALF)],
                o2.at[b], o2_sems.at[b],
            )
            c.start()
            o2_copies.append(c)

        barrier = pltpu.get_barrier_semaphore()
        for nbr in (ypartner, xneighbor, zneighbor):
            pl.semaphore_signal(
                barrier, inc=1, device_id=nbr,
                device_id_type=pl.DeviceIdType.MESH,
            )
        pl.semaphore_wait(barrier, 3)

        wo_copy.wait()

        q_mine = (2 * my_x + my_z) * QR
        q_x = (2 * (1 - my_x) + my_z) * QR
        q_z = (2 * my_x + (1 - my_z)) * QR
        q_d = (2 * (1 - my_x) + (1 - my_z)) * QR

        def _dot(src_ref, b):
            return lax.dot_general(
                src_ref[b], wo_ref[:, :],
                (((0,), (0,)), ((), ())),
                preferred_element_type=jnp.float32,
            )

        y_rdmas = []
        for b in range(B):
            o1_copies[b].wait()
            part = _dot(o1, b)
            ysend[b] = part.astype(jnp.bfloat16)
            r = pltpu.make_async_remote_copy(
                src_ref=ysend.at[b, pl.ds(my_z * QR, QR), :],
                dst_ref=yrecv.at[b],
                send_sem=ys_sems.at[b],
                recv_sem=yr_sems.at[b],
                device_id=ypartner,
                device_id_type=pl.DeviceIdType.MESH,
            )
            r.start()
            y_rdmas.append(r)

        x_rdmas, z_rdmas = [], []
        for b in range(B):
            o2_copies[b].wait()
            acc[b] = _dot(o2, b)
            y_rdmas[b].wait_recv()
            xr = pltpu.make_async_remote_copy(
                src_ref=yrecv.at[b], dst_ref=xrecv.at[b],
                send_sem=xs_sems.at[b], recv_sem=xr_sems.at[b],
                device_id=xneighbor, device_id_type=pl.DeviceIdType.MESH,
            )
            xr.start()
            x_rdmas.append(xr)
            zr = pltpu.make_async_remote_copy(
                src_ref=yrecv.at[b], dst_ref=zrecv.at[b],
                send_sem=zs_sems.at[b], recv_sem=zr_sems.at[b],
                device_id=zneighbor, device_id_type=pl.DeviceIdType.MESH,
            )
            zr.start()
            z_rdmas.append(zr)
            acc[b, pl.ds(q_mine, QR), :] = (
                acc[b, pl.ds(q_mine, QR), :] + yrecv[b].astype(jnp.float32)
            )

        d_rdmas = []
        for b in range(B):
            x_rdmas[b].wait_recv()
            z_rdmas[b].wait_recv()
            if b % 2 == 0:
                dr = pltpu.make_async_remote_copy(
                    src_ref=zrecv.at[b], dst_ref=drecv.at[b],
                    send_sem=ds_sems.at[b], recv_sem=dr_sems.at[b],
                    device_id=xneighbor, device_id_type=pl.DeviceIdType.MESH,
                )
            else:
                dr = pltpu.make_async_remote_copy(
                    src_ref=xrecv.at[b], dst_ref=drecv.at[b],
                    send_sem=ds_sems.at[b], recv_sem=dr_sems.at[b],
                    device_id=zneighbor, device_id_type=pl.DeviceIdType.MESH,
                )
            dr.start()
            d_rdmas.append(dr)
            acc[b, pl.ds(q_x, QR), :] = (
                acc[b, pl.ds(q_x, QR), :] + xrecv[b].astype(jnp.float32)
            )
            acc[b, pl.ds(q_z, QR), :] = (
                acc[b, pl.ds(q_z, QR), :] + zrecv[b].astype(jnp.float32)
            )

        out_copies = []
        for b in range(B):
            d_rdmas[b].wait_recv()
            acc[b, pl.ds(q_d, QR), :] = (
                acc[b, pl.ds(q_d, QR), :] + drecv[b].astype(jnp.float32)
            )
            oc = pltpu.make_async_copy(acc.at[b], out_hbm.at[b], out_sems.at[b])
            oc.start()
            out_copies.append(oc)

        for c in out_copies:
            c.wait()
        for b in range(B):
            y_rdmas[b].wait_send()
            x_rdmas[b].wait_send()
            z_rdmas[b].wait_send()
            d_rdmas[b].wait_send()

    out = pl.pallas_call(
        body,
        out_shape=jax.ShapeDtypeStruct((B, S_HALF, N), jnp.float32),
        in_specs=[
            pl.BlockSpec(memory_space=pltpu.MemorySpace.HBM),
            pl.BlockSpec(memory_space=pltpu.MemorySpace.HBM),
        ],
        out_specs=pl.BlockSpec(memory_space=pltpu.MemorySpace.HBM),
        scratch_shapes=[
            pltpu.VMEM((K, N), jnp.float32),
            pltpu.VMEM((B, K, 2 * QR), jnp.float32),
            pltpu.VMEM((B, K, S_HALF), jnp.float32),
            pltpu.VMEM((B, S_HALF, N), jnp.float32),
            pltpu.VMEM((B, 2 * QR, N), jnp.bfloat16),
            pltpu.VMEM((B, QR, N), jnp.bfloat16),
            pltpu.VMEM((B, QR, N), jnp.bfloat16),
            pltpu.VMEM((B, QR, N), jnp.bfloat16),
            pltpu.VMEM((B, QR, N), jnp.bfloat16),
            pltpu.SemaphoreType.DMA,
            pltpu.SemaphoreType.DMA((B,)),
            pltpu.SemaphoreType.DMA((B,)),
            pltpu.SemaphoreType.DMA((B,)),
            pltpu.SemaphoreType.DMA((B,)),
            pltpu.SemaphoreType.DMA((B,)),
            pltpu.SemaphoreType.DMA((B,)),
            pltpu.SemaphoreType.DMA((B,)),
            pltpu.SemaphoreType.DMA((B,)),
            pltpu.SemaphoreType.DMA((B,)),
            pltpu.SemaphoreType.DMA((B,)),
            pltpu.SemaphoreType.DMA((B,)),
        ],
        compiler_params=pltpu.CompilerParams(collective_id=0),
    )(O_t, Wo)
    return pltpu.with_memory_space_constraint(out, pltpu.MemorySpace.HBM)
